# baseline (speedup 1.0000x reference)
import functools

import jax
import jax.numpy as jnp
from jax import lax
from jax.experimental import pallas as pl
from jax.experimental.pallas import tpu as pltpu
from jax.experimental.pallas import tpu_sc as plsc

D = 128
_EPS = 1e-5

_CARD0 = 0
_MASKED = 52
_LEVEL0 = 56
_RUN0 = 72
_TROWS = 96

_NC = 2
_NS = 16
_NW = _NC * _NS
_CH = 128
_CTX_G = 256


def _ln_rows(x, g, b):
    mu = jnp.mean(x, axis=-1, keepdims=True)
    var = jnp.mean((x - mu) ** 2, axis=-1, keepdims=True)
    return (x - mu) / jnp.sqrt(var + _EPS) * g + b


def _table_kernel(rank_ref, suit_ref, level_ref, pwt_ref, pb_ref,
                  rg_ref, rb_ref, hg_ref, hb_ref, out_ref, rep_ref):
    hg = hg_ref[0:1, :]
    hb = hb_ref[0:1, :]
    card = jnp.concatenate(
        [rank_ref[:, :] + suit_ref[s:s + 1, :] for s in range(4)], axis=0)
    card_ln = _ln_rows(card, hg, hb)
    masked = jnp.broadcast_to(hb, (4, D))
    level = level_ref[:, :]
    ii = lax.broadcasted_iota(jnp.int32, (20, D), 0)
    h = (ii // 4).astype(jnp.float32)
    d = (ii % 4).astype(jnp.float32)
    x = h * pwt_ref[0:1, :] + d * pwt_ref[1:2, :] + pb_ref[0:1, :]
    run_ln = _ln_rows(x, rg_ref[0:1, :], rb_ref[0:1, :])
    pad = jnp.zeros((4, D), jnp.float32)
    tbl = jnp.concatenate([card_ln, masked, level, run_ln, pad], axis=0)
    out_ref[:, :] = tbl
    for s in range(_NW):
        rep_ref[pl.ds(s * _TROWS, _TROWS), :] = tbl


def _build_table(rank_emb, suit_emb, level_emb, pwt, pb, rg, rb, hg, hb):
    return pl.pallas_call(
        _table_kernel,
        out_shape=(
            jax.ShapeDtypeStruct((_TROWS, D), jnp.float32),
            jax.ShapeDtypeStruct((_NW * _TROWS, D), jnp.float32),
        ),
    )(rank_emb, suit_emb, level_emb, pwt, pb, rg, rb, hg, hb)


def _ctx_kernel(xidx_ref, tbl_ref, out_ref):
    g, nt1, _ = out_ref.shape
    rows_n = g * nt1
    oh = (xidx_ref[:, :]
          == lax.broadcasted_iota(jnp.int32, (rows_n, _TROWS), 1))
    oh = jnp.where(oh, 1.0, 0.0)
    rows = jax.lax.dot_general(
        oh, tbl_ref[:, :], (((1,), (0,)), ((), ())),
        preferred_element_type=jnp.float32,
        precision=jax.lax.Precision.HIGHEST)
    out_ref[:, :, :] = rows.reshape(g, nt1, D)


@functools.cache
def _make_ctx(B, nt1):
    return pl.pallas_call(
        _ctx_kernel,
        grid=(B // _CTX_G,),
        in_specs=[
            pl.BlockSpec((_CTX_G * nt1, 1), lambda i: (i, 0)),
            pl.BlockSpec((_TROWS, D), lambda i: (0, 0)),
        ],
        out_specs=pl.BlockSpec((_CTX_G, nt1, D), lambda i: (i, 0, 0)),
        out_shape=jax.ShapeDtypeStruct((B, nt1, D), jnp.float32),
    )


@functools.cache
def _make_gather(hand_rows):
    hand_ch = hand_rows // (_NW * _CH)
    mesh = plsc.VectorSubcoreMesh(core_axis_name="c", subcore_axis_name="s")

    @functools.partial(
        pl.kernel,
        mesh=mesh,
        out_type=jax.ShapeDtypeStruct((hand_rows, D), jnp.float32),
        scratch_types=[
            pltpu.VMEM((hand_ch * _CH,), jnp.int32),
            pltpu.VMEM((2, _CH, D), jnp.float32),
            pltpu.SemaphoreType.DMA,
            pltpu.SemaphoreType.DMA,
            pltpu.SemaphoreType.DMA,
            pltpu.SemaphoreType.DMA,
        ],
    )
    def gather(table_hbm, cidx_hbm, hand_hbm, cidx_v, bufs, g0, g1, o0, o1):
        wid = lax.axis_index("s") * _NC + lax.axis_index("c")
        gsem = (g0, g1)
        osem = (o0, o1)
        pltpu.sync_copy(
            cidx_hbm.at[pl.ds(pl.multiple_of(wid * (hand_ch * _CH), 8),
                              hand_ch * _CH)], cidx_v)

        def run(idx_v, out_hbm, nch):
            base = wid * nch * _CH

            def out_slice(c):
                return out_hbm.at[
                    pl.ds(pl.multiple_of(base + c * _CH, 8), _CH)]

            def idx_slice(c):
                return idx_v.at[pl.ds(pl.multiple_of(c * _CH, 8), _CH)]

            def g_start(c, b):
                pltpu.async_copy(table_hbm.at[idx_slice(c)], bufs.at[b],
                                 gsem[b])

            def g_wait(c, b):
                pltpu.make_async_copy(table_hbm.at[idx_slice(c)], bufs.at[b],
                                      gsem[b]).wait()

            def s_start(c, b):
                pltpu.async_copy(bufs.at[b], out_slice(c), osem[b])

            def s_wait(c, b):
                pltpu.make_async_copy(bufs.at[b], out_slice(c),
                                      osem[b]).wait()

            g_start(0, 0)
            g_start(1, 1)
            g_wait(0, 0)
            s_start(0, 0)

            def body(g, carry):
                for u in range(2):
                    c = 1 + g * 2 + u
                    b = (1 + u) % 2
                    s_wait(c - 1, 1 - b)
                    g_start(c + 1, 1 - b)
                    g_wait(c, b)
                    s_start(c, b)
                return carry

            lax.fori_loop(0, (nch - 2) // 2, body, 0)
            c = nch - 1
            b = c % 2
            g_wait(c, b)
            s_start(c, b)
            s_wait(c - 1, 1 - b)
            s_wait(c, b)

        run(cidx_v, hand_hbm, hand_ch)

    return gather


def kernel(hand_card_ids, hand_card_mask, hands_remaining, discards_remaining,
           hand_levels, rank_emb, suit_emb, proj_w, proj_b,
           run_ln_g, run_ln_b, hand_ln_g, hand_ln_b, level_emb):
    B, H = hand_card_ids.shape
    NT = hand_levels.shape[1]
    row = lambda v: v.astype(jnp.float32).reshape(1, D)
    table, table_rep = _build_table(
        rank_emb.astype(jnp.float32), suit_emb.astype(jnp.float32),
        level_emb.astype(jnp.float32), proj_w.astype(jnp.float32).T,
        row(proj_b), row(run_ln_g), row(run_ln_b),
        row(hand_ln_g), row(hand_ln_b))

    ids = hand_card_ids.astype(jnp.int32)
    cidx = jnp.where(hand_card_mask, ids, _MASKED).astype(jnp.int32)
    cidx = cidx.reshape(B * H)
    cidx = cidx + _TROWS * (jnp.arange(B * H, dtype=jnp.int32)
                            // (B * H // _NW))
    run_idx = (_RUN0 + 4 * hands_remaining.astype(jnp.int32)
               + discards_remaining.astype(jnp.int32))
    xidx = jnp.concatenate(
        [hand_levels.astype(jnp.int32) + _LEVEL0, run_idx], axis=1)

    hand_flat = _make_gather(B * H)(table_rep, cidx)
    ctx_seq = _make_ctx(B, NT + 1)(xidx.reshape(B * (NT + 1), 1), table)
    hand_toks = hand_flat.reshape(B, H, D)
    mask = hand_card_mask.astype(bool)
    ctx_mask = jnp.ones((B, NT + 1), dtype=bool)
    return hand_toks, mask, ctx_seq, ctx_mask


# speedup vs baseline: 1.1316x; 1.1316x over previous
import functools

import jax
import jax.numpy as jnp
from jax import lax
from jax.experimental import pallas as pl
from jax.experimental.pallas import tpu as pltpu
from jax.experimental.pallas import tpu_sc as plsc

D = 128
_EPS = 1e-5

_CARD0 = 0
_MASKED = 52
_LEVEL0 = 56
_RUN0 = 72
_TROWS = 96

_NC = 2
_NS = 16
_NW = _NC * _NS
_CH = 128
_CTX_G = 256


def _ln_rows(x, g, b):
    mu = jnp.mean(x, axis=-1, keepdims=True)
    var = jnp.mean((x - mu) ** 2, axis=-1, keepdims=True)
    return (x - mu) / jnp.sqrt(var + _EPS) * g + b


def _table_kernel(rank_ref, suit_ref, level_ref, pwt_ref, pb_ref,
                  rg_ref, rb_ref, hg_ref, hb_ref, out_ref, rep_ref):
    hg = hg_ref[0:1, :]
    hb = hb_ref[0:1, :]
    card = jnp.concatenate(
        [rank_ref[:, :] + suit_ref[s:s + 1, :] for s in range(4)], axis=0)
    card_ln = _ln_rows(card, hg, hb)
    masked = jnp.broadcast_to(hb, (4, D))
    level = level_ref[:, :]
    ii = lax.broadcasted_iota(jnp.int32, (20, D), 0)
    h = (ii // 4).astype(jnp.float32)
    d = (ii % 4).astype(jnp.float32)
    x = h * pwt_ref[0:1, :] + d * pwt_ref[1:2, :] + pb_ref[0:1, :]
    run_ln = _ln_rows(x, rg_ref[0:1, :], rb_ref[0:1, :])
    pad = jnp.zeros((4, D), jnp.float32)
    tbl = jnp.concatenate([card_ln, masked, level, run_ln, pad], axis=0)
    out_ref[:, :] = tbl
    for s in range(_NW):
        rep_ref[pl.ds(s * _TROWS, _TROWS), :] = tbl


def _build_table(rank_emb, suit_emb, level_emb, pwt, pb, rg, rb, hg, hb):
    return pl.pallas_call(
        _table_kernel,
        out_shape=(
            jax.ShapeDtypeStruct((_TROWS, D), jnp.float32),
            jax.ShapeDtypeStruct((_NW * _TROWS, D), jnp.float32),
        ),
    )(rank_emb, suit_emb, level_emb, pwt, pb, rg, rb, hg, hb)


def _ctx_kernel(xidx_ref, tbl_ref, out_ref):
    g, nt1, _ = out_ref.shape
    rows_n = g * nt1
    oh = (xidx_ref[:, :]
          == lax.broadcasted_iota(jnp.int32, (rows_n, _TROWS), 1))
    oh = jnp.where(oh, 1.0, 0.0)
    rows = jax.lax.dot_general(
        oh, tbl_ref[:, :], (((1,), (0,)), ((), ())),
        preferred_element_type=jnp.float32,
        precision=jax.lax.Precision.DEFAULT)
    out_ref[:, :, :] = rows.reshape(g, nt1, D)


@functools.cache
def _make_ctx(B, nt1):
    return pl.pallas_call(
        _ctx_kernel,
        grid=(B // _CTX_G,),
        in_specs=[
            pl.BlockSpec((_CTX_G * nt1, 1), lambda i: (i, 0)),
            pl.BlockSpec((_TROWS, D), lambda i: (0, 0)),
        ],
        out_specs=pl.BlockSpec((_CTX_G, nt1, D), lambda i: (i, 0, 0)),
        out_shape=jax.ShapeDtypeStruct((B, nt1, D), jnp.float32),
    )


@functools.cache
def _make_gather(hand_rows):
    hand_ch = hand_rows // (_NW * _CH)
    mesh = plsc.VectorSubcoreMesh(core_axis_name="c", subcore_axis_name="s")

    @functools.partial(
        pl.kernel,
        mesh=mesh,
        out_type=jax.ShapeDtypeStruct((hand_rows, D), jnp.float32),
        scratch_types=[
            pltpu.VMEM((hand_ch * _CH,), jnp.int32),
            pltpu.VMEM((2, _CH, D), jnp.float32),
            pltpu.SemaphoreType.DMA,
            pltpu.SemaphoreType.DMA,
            pltpu.SemaphoreType.DMA,
            pltpu.SemaphoreType.DMA,
        ],
    )
    def gather(table_hbm, cidx_hbm, hand_hbm, cidx_v, bufs, g0, g1, o0, o1):
        wid = lax.axis_index("s") * _NC + lax.axis_index("c")
        gsem = (g0, g1)
        osem = (o0, o1)
        pltpu.sync_copy(
            cidx_hbm.at[pl.ds(pl.multiple_of(wid * (hand_ch * _CH), 8),
                              hand_ch * _CH)], cidx_v)

        def run(idx_v, out_hbm, nch):
            base = wid * nch * _CH

            def out_slice(c):
                return out_hbm.at[
                    pl.ds(pl.multiple_of(base + c * _CH, 8), _CH)]

            def idx_slice(c):
                return idx_v.at[pl.ds(pl.multiple_of(c * _CH, 8), _CH)]

            def g_start(c, b):
                pltpu.async_copy(table_hbm.at[idx_slice(c)], bufs.at[b],
                                 gsem[b])

            def g_wait(c, b):
                pltpu.make_async_copy(table_hbm.at[idx_slice(c)], bufs.at[b],
                                      gsem[b]).wait()

            def s_start(c, b):
                pltpu.async_copy(bufs.at[b], out_slice(c), osem[b])

            def s_wait(c, b):
                pltpu.make_async_copy(bufs.at[b], out_slice(c),
                                      osem[b]).wait()

            g_start(0, 0)
            g_start(1, 1)
            g_wait(0, 0)
            s_start(0, 0)

            def body(g, carry):
                for u in range(2):
                    c = 1 + g * 2 + u
                    b = (1 + u) % 2
                    s_wait(c - 1, 1 - b)
                    g_start(c + 1, 1 - b)
                    g_wait(c, b)
                    s_start(c, b)
                return carry

            lax.fori_loop(0, (nch - 2) // 2, body, 0)
            c = nch - 1
            b = c % 2
            g_wait(c, b)
            s_start(c, b)
            s_wait(c - 1, 1 - b)
            s_wait(c, b)

        run(cidx_v, hand_hbm, hand_ch)

    return gather


def kernel(hand_card_ids, hand_card_mask, hands_remaining, discards_remaining,
           hand_levels, rank_emb, suit_emb, proj_w, proj_b,
           run_ln_g, run_ln_b, hand_ln_g, hand_ln_b, level_emb):
    B, H = hand_card_ids.shape
    NT = hand_levels.shape[1]
    row = lambda v: v.astype(jnp.float32).reshape(1, D)
    table, table_rep = _build_table(
        rank_emb.astype(jnp.float32), suit_emb.astype(jnp.float32),
        level_emb.astype(jnp.float32), proj_w.astype(jnp.float32).T,
        row(proj_b), row(run_ln_g), row(run_ln_b),
        row(hand_ln_g), row(hand_ln_b))

    ids = hand_card_ids.astype(jnp.int32)
    cidx = jnp.where(hand_card_mask, ids, _MASKED).astype(jnp.int32)
    cidx = cidx.reshape(B * H)
    cidx = cidx + _TROWS * (jnp.arange(B * H, dtype=jnp.int32)
                            // (B * H // _NW))
    run_idx = (_RUN0 + 4 * hands_remaining.astype(jnp.int32)
               + discards_remaining.astype(jnp.int32))
    xidx = jnp.concatenate(
        [hand_levels.astype(jnp.int32) + _LEVEL0, run_idx], axis=1)

    hand_flat = _make_gather(B * H)(table_rep, cidx)
    ctx_seq = _make_ctx(B, NT + 1)(xidx.reshape(B * (NT + 1), 1), table)
    hand_toks = hand_flat.reshape(B, H, D)
    mask = hand_card_mask.astype(bool)
    ctx_mask = jnp.ones((B, NT + 1), dtype=bool)
    return hand_toks, mask, ctx_seq, ctx_mask


# speedup vs baseline: 1.5043x; 1.3295x over previous
import functools

import jax
import jax.numpy as jnp
from jax import lax
from jax.experimental import pallas as pl
from jax.experimental.pallas import tpu as pltpu
from jax.experimental.pallas import tpu_sc as plsc

D = 128
_EPS = 1e-5

_CARD0 = 0
_MASKED = 52
_LEVEL0 = 56
_RUN0 = 72
_TROWS = 96

_NC = 2
_NS = 16
_NW = _NC * _NS
_CH = 128
_CTX_G = 256


def _ln_rows(x, g, b):
    mu = jnp.mean(x, axis=-1, keepdims=True)
    var = jnp.mean((x - mu) ** 2, axis=-1, keepdims=True)
    return (x - mu) / jnp.sqrt(var + _EPS) * g + b


def _table_kernel(rank_ref, suit_ref, level_ref, pwt_ref, pb_ref,
                  rg_ref, rb_ref, hg_ref, hb_ref, out_ref, rep_ref):
    hg = hg_ref[0:1, :]
    hb = hb_ref[0:1, :]
    card = jnp.concatenate(
        [rank_ref[:, :] + suit_ref[s:s + 1, :] for s in range(4)], axis=0)
    card_ln = _ln_rows(card, hg, hb)
    masked = jnp.broadcast_to(hb, (4, D))
    level = level_ref[:, :]
    ii = lax.broadcasted_iota(jnp.int32, (20, D), 0)
    h = (ii // 4).astype(jnp.float32)
    d = (ii % 4).astype(jnp.float32)
    x = h * pwt_ref[0:1, :] + d * pwt_ref[1:2, :] + pb_ref[0:1, :]
    run_ln = _ln_rows(x, rg_ref[0:1, :], rb_ref[0:1, :])
    pad = jnp.zeros((4, D), jnp.float32)
    tbl = jnp.concatenate([card_ln, masked, level, run_ln, pad], axis=0)
    out_ref[:, :] = tbl
    for s in range(_NW):
        rep_ref[pl.ds(s * _TROWS, _TROWS), :] = tbl


def _build_table(rank_emb, suit_emb, level_emb, pwt, pb, rg, rb, hg, hb):
    return pl.pallas_call(
        _table_kernel,
        out_shape=(
            jax.ShapeDtypeStruct((_TROWS, D), jnp.float32),
            jax.ShapeDtypeStruct((_NW * _TROWS, D), jnp.float32),
        ),
    )(rank_emb, suit_emb, level_emb, pwt, pb, rg, rb, hg, hb)


def _ctx_kernel(xidx_ref, tblt_ref, out_ref):
    g, nt1, _ = out_ref.shape
    nk = xidx_ref.shape[1]
    tblt = tblt_ref[:, :]
    sub_iota = lax.broadcasted_iota(jnp.int32, (_TROWS, D), 0)
    chunks = []
    for k in range(nk):
        oht = jnp.where(
            sub_iota == jnp.broadcast_to(xidx_ref[0, k:k + 1, :],
                                         (_TROWS, D)),
            1.0, 0.0)
        rows_t = jax.lax.dot_general(
            tblt, oht, (((1,), (0,)), ((), ())),
            preferred_element_type=jnp.float32,
            precision=jax.lax.Precision.DEFAULT)
        chunks.append(rows_t.T)
    rows = jnp.concatenate(chunks, axis=0)
    out_ref[:, :, :] = rows.reshape(g, nt1, D)


@functools.cache
def _make_ctx(B, nt1):
    nk = _CTX_G * nt1 // D
    return pl.pallas_call(
        _ctx_kernel,
        grid=(B // _CTX_G,),
        in_specs=[
            pl.BlockSpec((1, nk, D), lambda i: (i, 0, 0)),
            pl.BlockSpec((D, _TROWS), lambda i: (0, 0)),
        ],
        out_specs=pl.BlockSpec((_CTX_G, nt1, D), lambda i: (i, 0, 0)),
        out_shape=jax.ShapeDtypeStruct((B, nt1, D), jnp.float32),
    )


@functools.cache
def _make_gather(hand_rows):
    hand_ch = hand_rows // (_NW * _CH)
    mesh = plsc.VectorSubcoreMesh(core_axis_name="c", subcore_axis_name="s")

    @functools.partial(
        pl.kernel,
        mesh=mesh,
        out_type=jax.ShapeDtypeStruct((hand_rows, D), jnp.float32),
        scratch_types=[
            pltpu.VMEM((hand_ch * _CH,), jnp.int32),
            pltpu.VMEM((2, _CH, D), jnp.float32),
            pltpu.SemaphoreType.DMA,
            pltpu.SemaphoreType.DMA,
            pltpu.SemaphoreType.DMA,
            pltpu.SemaphoreType.DMA,
        ],
    )
    def gather(table_hbm, cidx_hbm, hand_hbm, cidx_v, bufs, g0, g1, o0, o1):
        wid = lax.axis_index("s") * _NC + lax.axis_index("c")
        gsem = (g0, g1)
        osem = (o0, o1)
        pltpu.sync_copy(
            cidx_hbm.at[pl.ds(pl.multiple_of(wid * (hand_ch * _CH), 8),
                              hand_ch * _CH)], cidx_v)

        def run(idx_v, out_hbm, nch):
            base = wid * nch * _CH

            def out_slice(c):
                return out_hbm.at[
                    pl.ds(pl.multiple_of(base + c * _CH, 8), _CH)]

            def idx_slice(c):
                return idx_v.at[pl.ds(pl.multiple_of(c * _CH, 8), _CH)]

            def g_start(c, b):
                pltpu.async_copy(table_hbm.at[idx_slice(c)], bufs.at[b],
                                 gsem[b])

            def g_wait(c, b):
                pltpu.make_async_copy(table_hbm.at[idx_slice(c)], bufs.at[b],
                                      gsem[b]).wait()

            def s_start(c, b):
                pltpu.async_copy(bufs.at[b], out_slice(c), osem[b])

            def s_wait(c, b):
                pltpu.make_async_copy(bufs.at[b], out_slice(c),
                                      osem[b]).wait()

            g_start(0, 0)
            g_start(1, 1)
            g_wait(0, 0)
            s_start(0, 0)

            def body(g, carry):
                for u in range(2):
                    c = 1 + g * 2 + u
                    b = (1 + u) % 2
                    s_wait(c - 1, 1 - b)
                    g_start(c + 1, 1 - b)
                    g_wait(c, b)
                    s_start(c, b)
                return carry

            lax.fori_loop(0, (nch - 2) // 2, body, 0)
            c = nch - 1
            b = c % 2
            g_wait(c, b)
            s_start(c, b)
            s_wait(c - 1, 1 - b)
            s_wait(c, b)

        run(cidx_v, hand_hbm, hand_ch)

    return gather


def kernel(hand_card_ids, hand_card_mask, hands_remaining, discards_remaining,
           hand_levels, rank_emb, suit_emb, proj_w, proj_b,
           run_ln_g, run_ln_b, hand_ln_g, hand_ln_b, level_emb):
    B, H = hand_card_ids.shape
    NT = hand_levels.shape[1]
    row = lambda v: v.astype(jnp.float32).reshape(1, D)
    table, table_rep = _build_table(
        rank_emb.astype(jnp.float32), suit_emb.astype(jnp.float32),
        level_emb.astype(jnp.float32), proj_w.astype(jnp.float32).T,
        row(proj_b), row(run_ln_g), row(run_ln_b),
        row(hand_ln_g), row(hand_ln_b))

    ids = hand_card_ids.astype(jnp.int32)
    cidx = jnp.where(hand_card_mask, ids, _MASKED).astype(jnp.int32)
    cidx = cidx.reshape(B * H)
    cidx = cidx + _TROWS * (jnp.arange(B * H, dtype=jnp.int32)
                            // (B * H // _NW))
    run_idx = (_RUN0 + 4 * hands_remaining.astype(jnp.int32)
               + discards_remaining.astype(jnp.int32))
    xidx = jnp.concatenate(
        [hand_levels.astype(jnp.int32) + _LEVEL0, run_idx], axis=1)

    hand_flat = _make_gather(B * H)(table_rep, cidx)
    nk = _CTX_G * (NT + 1) // D
    ctx_seq = _make_ctx(B, NT + 1)(
        xidx.reshape(B // _CTX_G, nk, D), table.T)
    hand_toks = hand_flat.reshape(B, H, D)
    mask = hand_card_mask.astype(bool)
    ctx_mask = jnp.ones((B, NT + 1), dtype=bool)
    return hand_toks, mask, ctx_seq, ctx_mask


# speedup vs baseline: 4.1469x; 2.7566x over previous
import functools

import jax
import jax.numpy as jnp
from jax import lax
from jax.experimental import pallas as pl
from jax.experimental.pallas import tpu as pltpu
from jax.experimental.pallas import tpu_sc as plsc

D = 128
_EPS = 1e-5

_CARD0 = 0
_MASKED = 52
_LEVEL0 = 56
_RUN0 = 72
_TROWS = 96

_NC = 2
_NS = 16
_NW = _NC * _NS
_CH = 128
_CTX_G = 1024


def _ln_rows(x, g, b):
    mu = jnp.mean(x, axis=-1, keepdims=True)
    var = jnp.mean((x - mu) ** 2, axis=-1, keepdims=True)
    return (x - mu) / jnp.sqrt(var + _EPS) * g + b


def _table_kernel(rank_ref, suit_ref, level_ref, pwt_ref, pb_ref,
                  rg_ref, rb_ref, hg_ref, hb_ref, out_ref):
    hg = hg_ref[0:1, :]
    hb = hb_ref[0:1, :]
    card = jnp.concatenate(
        [rank_ref[:, :] + suit_ref[s:s + 1, :] for s in range(4)], axis=0)
    card_ln = _ln_rows(card, hg, hb)
    masked = jnp.broadcast_to(hb, (4, D))
    level = level_ref[:, :]
    ii = lax.broadcasted_iota(jnp.int32, (20, D), 0)
    h = (ii // 4).astype(jnp.float32)
    d = (ii % 4).astype(jnp.float32)
    x = h * pwt_ref[0:1, :] + d * pwt_ref[1:2, :] + pb_ref[0:1, :]
    run_ln = _ln_rows(x, rg_ref[0:1, :], rb_ref[0:1, :])
    pad = jnp.zeros((4, D), jnp.float32)
    tbl = jnp.concatenate([card_ln, masked, level, run_ln, pad], axis=0)
    out_ref[:, :] = tbl


def _build_table(rank_emb, suit_emb, level_emb, pwt, pb, rg, rb, hg, hb):
    return pl.pallas_call(
        _table_kernel,
        out_shape=jax.ShapeDtypeStruct((_TROWS, D), jnp.float32),
    )(rank_emb, suit_emb, level_emb, pwt, pb, rg, rb, hg, hb)


def _ctx_kernel(xidxt_ref, tblt_ref, out_ref):
    nt1, g, _ = out_ref.shape
    tblt = tblt_ref[:, :]
    sub_iota = lax.broadcasted_iota(jnp.int32, (_TROWS, D), 0)
    for j in range(nt1):
        for c in range(g // D):
            idx_chunk = xidxt_ref[j:j + 1, c * D:(c + 1) * D]
            oht = jnp.where(
                sub_iota == jnp.broadcast_to(idx_chunk, (_TROWS, D)),
                1.0, 0.0)
            rows_t = jax.lax.dot_general(
                tblt, oht, (((1,), (0,)), ((), ())),
                preferred_element_type=jnp.float32,
                precision=jax.lax.Precision.DEFAULT)
            out_ref[j, pl.ds(c * D, D), :] = rows_t.T


@functools.cache
def _make_ctx(B, nt1):
    return pl.pallas_call(
        _ctx_kernel,
        grid=(B // _CTX_G,),
        in_specs=[
            pl.BlockSpec((nt1, _CTX_G), lambda i: (0, i)),
            pl.BlockSpec((D, _TROWS), lambda i: (0, 0)),
        ],
        out_specs=pl.BlockSpec((nt1, _CTX_G, D), lambda i: (0, i, 0)),
        out_shape=jax.ShapeDtypeStruct((nt1, B, D), jnp.float32),
    )


@functools.cache
def _make_gather(hand_rows):
    hand_ch = hand_rows // (_NW * _CH)
    mesh = plsc.VectorSubcoreMesh(core_axis_name="c", subcore_axis_name="s")

    @functools.partial(
        pl.kernel,
        mesh=mesh,
        out_type=jax.ShapeDtypeStruct((hand_rows, D), jnp.float32),
        scratch_types=[
            pltpu.VMEM((hand_ch * _CH,), jnp.int32),
            pltpu.VMEM_SHARED((_TROWS, D), jnp.float32),
            pltpu.VMEM((2, _CH, D), jnp.float32),
            pltpu.SemaphoreType.DMA,
            pltpu.SemaphoreType.DMA,
            pltpu.SemaphoreType.DMA,
            pltpu.SemaphoreType.DMA,
        ],
    )
    def gather(table_hbm, cidx_hbm, hand_hbm, cidx_v, tbl_v, bufs,
               g0, g1, o0, o1):
        wid = lax.axis_index("s") * _NC + lax.axis_index("c")
        gsem = (g0, g1)
        osem = (o0, o1)

        @pl.when(lax.axis_index("s") == 0)
        def _load_table():
            pltpu.sync_copy(table_hbm, tbl_v)

        plsc.subcore_barrier()
        pltpu.sync_copy(
            cidx_hbm.at[pl.ds(pl.multiple_of(wid * (hand_ch * _CH), 8),
                              hand_ch * _CH)], cidx_v)

        def run(idx_v, out_hbm, nch):
            base = wid * nch * _CH

            def out_slice(c):
                return out_hbm.at[
                    pl.ds(pl.multiple_of(base + c * _CH, 8), _CH)]

            def idx_slice(c):
                return idx_v.at[pl.ds(pl.multiple_of(c * _CH, 8), _CH)]

            def g_start(c, b):
                pltpu.async_copy(tbl_v.at[idx_slice(c)], bufs.at[b],
                                 gsem[b])

            def g_wait(c, b):
                pltpu.make_async_copy(tbl_v.at[idx_slice(c)], bufs.at[b],
                                      gsem[b]).wait()

            def s_start(c, b):
                pltpu.async_copy(bufs.at[b], out_slice(c), osem[b])

            def s_wait(c, b):
                pltpu.make_async_copy(bufs.at[b], out_slice(c),
                                      osem[b]).wait()

            g_start(0, 0)
            g_start(1, 1)
            g_wait(0, 0)
            s_start(0, 0)

            def body(g, carry):
                for u in range(2):
                    c = 1 + g * 2 + u
                    b = (1 + u) % 2
                    s_wait(c - 1, 1 - b)
                    g_start(c + 1, 1 - b)
                    g_wait(c, b)
                    s_start(c, b)
                return carry

            lax.fori_loop(0, (nch - 2) // 2, body, 0)
            c = nch - 1
            b = c % 2
            g_wait(c, b)
            s_start(c, b)
            s_wait(c - 1, 1 - b)
            s_wait(c, b)

        run(cidx_v, hand_hbm, hand_ch)

    return gather


def kernel(hand_card_ids, hand_card_mask, hands_remaining, discards_remaining,
           hand_levels, rank_emb, suit_emb, proj_w, proj_b,
           run_ln_g, run_ln_b, hand_ln_g, hand_ln_b, level_emb):
    B, H = hand_card_ids.shape
    NT = hand_levels.shape[1]
    row = lambda v: v.astype(jnp.float32).reshape(1, D)
    table = _build_table(
        rank_emb.astype(jnp.float32), suit_emb.astype(jnp.float32),
        level_emb.astype(jnp.float32), proj_w.astype(jnp.float32).T,
        row(proj_b), row(run_ln_g), row(run_ln_b),
        row(hand_ln_g), row(hand_ln_b))

    ids = hand_card_ids.astype(jnp.int32)
    cidx = jnp.where(hand_card_mask, ids, _MASKED).astype(jnp.int32)
    cidx = cidx.reshape(B * H)
    run_idx = (_RUN0 + 4 * hands_remaining.astype(jnp.int32)
               + discards_remaining.astype(jnp.int32))
    xidxt = jnp.concatenate(
        [hand_levels.astype(jnp.int32).T + _LEVEL0, run_idx.T], axis=0)

    hand_flat = _make_gather(B * H)(table, cidx)
    ctx_planes = _make_ctx(B, NT + 1)(xidxt, table.T)
    ctx_seq = ctx_planes.transpose(1, 0, 2)
    hand_toks = hand_flat.reshape(B, H, D)
    mask = hand_card_mask.astype(bool)
    ctx_mask = jnp.ones((B, NT + 1), dtype=bool)
    return hand_toks, mask, ctx_seq, ctx_mask
